# Initial kernel scaffold; baseline (speedup 1.0000x reference)
import jax
import jax.numpy as jnp
from jax.experimental import pallas as pl


def kernel(x, edge_index, batch, params):
    raise NotImplementedError("write your pallas kernel here")


# speedup vs baseline: 12.2879x; 12.2879x over previous
import functools

import jax
import jax.numpy as jnp
from jax import lax
from jax.experimental import pallas as pl
from jax.experimental.pallas import tpu as pltpu
from jax.experimental.pallas import tpu_sc as plsc

N = 10000
D = 128
H = 64
G = 128
C = 10
E = 320000

NC = 2
NS = 16
NW = NC * NS
CH = 128
CPW = 80
E_PAD = NW * CPW * CH
NACC = 10112
RPT = NACC // NS
ZB = 79

BLK = 1000
NBLK = N // BLK


def _sc_segment_sum(p, src3, dst3):
    mesh = plsc.VectorSubcoreMesh(core_axis_name="c", subcore_axis_name="s")

    @functools.partial(
        pl.kernel,
        out_type=jax.ShapeDtypeStruct((NC, NACC, H), jnp.float32),
        mesh=mesh,
        scratch_types=[
            pltpu.VMEM((CPW, CH), jnp.int32),
            pltpu.VMEM((CPW, CH), jnp.int32),
            pltpu.VMEM((CH, H), jnp.float32),
            pltpu.VMEM((CH, H), jnp.float32),
            pltpu.VMEM((ZB, H), jnp.float32),
            pltpu.VMEM_SHARED((NACC, H), jnp.float32),
            pltpu.SemaphoreType.DMA,
            pltpu.SemaphoreType.DMA,
        ],
        compiler_params=pltpu.CompilerParams(use_tc_tiling_on_sc=False),
    )
    def agg(p_hbm, src_hbm, dst_hbm, out_hbm,
            src_v, dst_v, rows_a, rows_b, zbuf, acc, sem_a, sem_b):
        c = lax.axis_index("c")
        s = lax.axis_index("s")
        wid = c * NS + s

        pltpu.sync_copy(src_hbm.at[wid], src_v)
        pltpu.sync_copy(dst_hbm.at[wid], dst_v)

        def _zrow(i, _):
            for l in range(H // 16):
                zbuf[i, pl.ds(l * 16, 16)] = jnp.zeros((16,), jnp.float32)
            return 0
        lax.fori_loop(0, ZB, _zrow, 0)
        base = s * RPT
        for r in range(RPT // ZB):
            pltpu.sync_copy(zbuf, acc.at[pl.ds(base + r * ZB, ZB)])
        plsc.subcore_barrier()

        pltpu.async_copy(p_hbm.at[src_v.at[0]], rows_a, sem_a)

        def _pair(t, _):
            j = 2 * t
            pltpu.async_copy(p_hbm.at[src_v.at[j + 1]], rows_b, sem_b)
            pltpu.make_async_copy(p_hbm.at[src_v.at[0]], rows_a, sem_a).wait()
            pltpu.sync_copy(rows_a, acc.at[dst_v.at[j]], add=True)

            @pl.when(j + 2 < CPW)
            def _():
                pltpu.async_copy(p_hbm.at[src_v.at[j + 2]], rows_a, sem_a)

            pltpu.make_async_copy(p_hbm.at[src_v.at[0]], rows_b, sem_b).wait()
            pltpu.sync_copy(rows_b, acc.at[dst_v.at[j + 1]], add=True)
            return 0

        lax.fori_loop(0, CPW // 2, _pair, 0)
        plsc.subcore_barrier()

        pltpu.sync_copy(acc.at[pl.ds(base, RPT)],
                        out_hbm.at[c, pl.ds(base, RPT)])

    return agg(p, src3, dst3)


def _proj_body(x_ref, w_ref, o_ref):
    o_ref[...] = jnp.dot(x_ref[...], w_ref[...],
                         preferred_element_type=jnp.float32)


def _tc_project(x, w):
    d_in = x.shape[1]
    return pl.pallas_call(
        _proj_body,
        grid=(NBLK,),
        in_specs=[
            pl.BlockSpec((BLK, d_in), lambda i: (i, 0)),
            pl.BlockSpec((d_in, H), lambda i: (0, 0)),
        ],
        out_specs=pl.BlockSpec((BLK, H), lambda i: (i, 0)),
        out_shape=jax.ShapeDtypeStruct((N, H), jnp.float32),
    )(x, w)


def _mlp_core(pa_ref, pb_ref, p_ref, eps_ref, b1_ref, w2_ref, b2_ref,
              g_ref, be_ref):
    h1 = pa_ref[...] + pb_ref[...] + eps_ref[0, 0] * p_ref[...] + b1_ref[...]
    h1 = jnp.maximum(h1, 0.0)
    h2 = jnp.dot(h1, w2_ref[...], preferred_element_type=jnp.float32)
    h2 = jnp.maximum(h2 + b2_ref[...], 0.0)
    return jnp.maximum(h2 * g_ref[...] + be_ref[...], 0.0)


def _mlp_body(pa_ref, pb_ref, p_ref, eps_ref, b1_ref, w2_ref, b2_ref,
              g_ref, be_ref, w1n_ref, xn_ref, pn_ref):
    xn = _mlp_core(pa_ref, pb_ref, p_ref, eps_ref, b1_ref, w2_ref, b2_ref,
                   g_ref, be_ref)
    xn_ref[...] = xn
    pn_ref[...] = jnp.dot(xn, w1n_ref[...], preferred_element_type=jnp.float32)


def _mlp_pool_body(pa_ref, pb_ref, p_ref, eps_ref, b1_ref, w2_ref, b2_ref,
                   g_ref, be_ref, batch_ref, xn_ref, pooled_ref):
    xn = _mlp_core(pa_ref, pb_ref, p_ref, eps_ref, b1_ref, w2_ref, b2_ref,
                   g_ref, be_ref)
    xn_ref[...] = xn
    b = batch_ref[0, 0, :]
    oh = (b[None, :] == lax.broadcasted_iota(jnp.int32, (G, BLK), 0))
    oh = oh.astype(jnp.float32)
    aug = jnp.concatenate(
        [xn, jnp.ones((BLK, 1), jnp.float32)], axis=1)

    @pl.when(pl.program_id(0) == 0)
    def _():
        pooled_ref[...] = jnp.zeros_like(pooled_ref)

    pooled_ref[...] += jnp.dot(oh, aug, preferred_element_type=jnp.float32)


def _row_specs(d_in):
    return [
        pl.BlockSpec((BLK, d_in), lambda i: (i, 0)),
        pl.BlockSpec((BLK, d_in), lambda i: (i, 0)),
        pl.BlockSpec((BLK, d_in), lambda i: (i, 0)),
        pl.BlockSpec((1, 1), lambda i: (0, 0)),
        pl.BlockSpec((1, H), lambda i: (0, 0)),
        pl.BlockSpec((H, H), lambda i: (0, 0)),
        pl.BlockSpec((1, H), lambda i: (0, 0)),
        pl.BlockSpec((1, H), lambda i: (0, 0)),
        pl.BlockSpec((1, H), lambda i: (0, 0)),
    ]


def _tc_mlp(parts, p, cp, w1n):
    return pl.pallas_call(
        _mlp_body,
        grid=(NBLK,),
        in_specs=_row_specs(H) + [pl.BlockSpec((H, H), lambda i: (0, 0))],
        out_specs=[
            pl.BlockSpec((BLK, H), lambda i: (i, 0)),
            pl.BlockSpec((BLK, H), lambda i: (i, 0)),
        ],
        out_shape=[
            jax.ShapeDtypeStruct((N, H), jnp.float32),
            jax.ShapeDtypeStruct((N, H), jnp.float32),
        ],
    )(parts[0], parts[1], p,
      (1.0 + cp['eps']).reshape(1, 1), cp['b1'].reshape(1, H), cp['W2'],
      cp['b2'].reshape(1, H), cp['gamma'].reshape(1, H),
      cp['beta'].reshape(1, H), w1n)


def _tc_mlp_pool(parts, p, cp, batch2d):
    return pl.pallas_call(
        _mlp_pool_body,
        grid=(NBLK,),
        in_specs=_row_specs(H) + [pl.BlockSpec((1, 1, BLK),
                                               lambda i: (i, 0, 0))],
        out_specs=[
            pl.BlockSpec((BLK, H), lambda i: (i, 0)),
            pl.BlockSpec((G, H + 1), lambda i: (0, 0)),
        ],
        out_shape=[
            jax.ShapeDtypeStruct((N, H), jnp.float32),
            jax.ShapeDtypeStruct((G, H + 1), jnp.float32),
        ],
    )(parts[0], parts[1], p,
      (1.0 + cp['eps']).reshape(1, 1), cp['b1'].reshape(1, H), cp['W2'],
      cp['b2'].reshape(1, H), cp['gamma'].reshape(1, H),
      cp['beta'].reshape(1, H), batch2d)


def _heads_body(pooled_ref, w1a_ref, b1a_ref, w2a_ref, b2a_ref,
                w1b_ref, b1b_ref, w2b_ref, b2b_ref,
                h_ref, ya_ref, yb_ref):
    sums = pooled_ref[:, :H]
    cnt = pooled_ref[:, H:H + 1]
    h = sums / jnp.maximum(cnt, 1.0)
    h_ref[...] = h
    za = jnp.maximum(jnp.dot(h, w1a_ref[...],
                             preferred_element_type=jnp.float32)
                     + b1a_ref[...], 0.0)
    ya_ref[...] = jnp.dot(za, w2a_ref[...],
                          preferred_element_type=jnp.float32) + b2a_ref[...]
    zb = jnp.maximum(jnp.dot(h, w1b_ref[...],
                             preferred_element_type=jnp.float32)
                     + b1b_ref[...], 0.0)
    yb_ref[...] = jnp.dot(zb, w2b_ref[...],
                          preferred_element_type=jnp.float32) + b2b_ref[...]


def _tc_heads(pooled, lin1, lin2):
    return pl.pallas_call(
        _heads_body,
        out_shape=[
            jax.ShapeDtypeStruct((G, H), jnp.float32),
            jax.ShapeDtypeStruct((G, C), jnp.float32),
            jax.ShapeDtypeStruct((G, C), jnp.float32),
        ],
    )(pooled,
      lin1[0]['W'], lin1[0]['b'].reshape(1, H),
      lin2[0]['W'], lin2[0]['b'].reshape(1, C),
      lin1[1]['W'], lin1[1]['b'].reshape(1, H),
      lin2[1]['W'], lin2[1]['b'].reshape(1, C))


def kernel(x, edge_index, batch, params):
    pad = E_PAD - E
    pad_i = jnp.arange(pad, dtype=jnp.int32)
    src3 = jnp.concatenate([edge_index[0], pad_i % N]).reshape(NW, CPW, CH)
    dst3 = jnp.concatenate([edge_index[1], N + pad_i % (NACC - N)]
                           ).reshape(NW, CPW, CH)
    batch2d = batch.reshape(NBLK, 1, BLK)

    convs = params['convs']
    p = _tc_project(x, convs[0]['W1'])
    xss = []
    for i in range(3):
        parts = _sc_segment_sum(p, src3, dst3)
        if i < 2:
            xn, p = _tc_mlp(parts, p, convs[i], convs[i + 1]['W1'])
        else:
            xn, pooled = _tc_mlp_pool(parts, p, convs[i], batch2d)
        xss.append(xn)

    h, y0, y1 = _tc_heads(pooled, params['lin1'], params['lin2'])
    return (edge_index, xss[0], xss[1], xss[2], h, h, y0, y1)
